# baseline (speedup 1.0000x reference)
import functools

import jax
import jax.numpy as jnp
from jax import lax
from jax.experimental import pallas as pl
from jax.experimental.pallas import tpu as pltpu
from jax.experimental.pallas import tpu_sc as plsc

_N = 10000
_E = 320000
_LAT = 128
_STEPS = 10
_RADIUS = 0.015
_NTYPES = 9
_EMB = 16

_NC = 2
_NS = 16
_NW = _NC * _NS
_CHUNK = 128
_CH = 80
_CHH = _CH // 2
_BLK = 2
_EPAD = _NW * _CH * _CHUNK
_NPAD = 10112
_RPT = _NPAD // _NS

_RB = 2000
_GRID = _N // _RB

_sc_mesh = plsc.VectorSubcoreMesh(core_axis_name="c", subcore_axis_name="s",
                                  num_cores=_NC, num_subcores=_NS)


def _make_sc_aggregate(ch):
    @functools.partial(
        pl.kernel,
        out_type=jax.ShapeDtypeStruct((_NC, _NPAD, _LAT), jnp.float32),
        mesh=_sc_mesh,
        scratch_types=[
            pltpu.VMEM((_CHH * _CHUNK,), jnp.int32),
            pltpu.VMEM((_CHH * _CHUNK,), jnp.int32),
            pltpu.VMEM((_BLK * _CHUNK, _LAT), jnp.float32),
            pltpu.VMEM_SHARED((_NPAD, _LAT), jnp.float32),
            pltpu.SemaphoreType.DMA,
        ],
    )
    def aggregate(h_hbm, sgrp_hbm, rgrp_hbm, zeros_hbm, out_hbm,
                  sidx_v, ridx_v, rows_v, acc_sh, gsem):
        c = lax.axis_index("c")
        s = lax.axis_index("s")
        g = c * _NS + s
        row0 = s * _RPT
        pltpu.sync_copy(zeros_hbm.at[pl.ds(row0, _RPT)],
                        acc_sh.at[pl.ds(row0, _RPT)])
        plsc.subcore_barrier()

        for hh in range(2):
            pltpu.sync_copy(sgrp_hbm.at[g, hh], sidx_v)
            pltpu.sync_copy(rgrp_hbm.at[g, hh], ridx_v)

            def body(i, carry):
                j0 = i * _BLK * _CHUNK
                pltpu.async_copy(
                    h_hbm.at[sidx_v.at[pl.ds(j0, _BLK * _CHUNK)]],
                    rows_v, gsem).wait()
                pltpu.sync_copy(
                    rows_v, acc_sh.at[ridx_v.at[pl.ds(j0, _BLK * _CHUNK)]],
                    add=True)
                return carry

            lax.fori_loop(0, _CHH // _BLK, body, 0)
        plsc.subcore_barrier()
        pltpu.sync_copy(acc_sh.at[pl.ds(row0, _RPT)],
                        out_hbm.at[c, pl.ds(row0, _RPT)])

    return aggregate


_sc_aggregate = _make_sc_aggregate(_CH)


def _layernorm(x):
    m = jnp.mean(x, axis=-1, keepdims=True)
    d = x - m
    v = jnp.mean(d * d, axis=-1, keepdims=True)
    return d * lax.rsqrt(v + 1e-6)


def _enc_body(pos_ref, pt_ref, emb_ref, w1_ref, b1_ref, w2_ref, b2_ref, h_ref):
    x = pos_ref[...]
    mr = x[:, 15:18]
    vel = x[:, 3:18] - x[:, 0:15]
    d_low = mr - 0.1
    d_up = 0.9 - mr
    nd = jnp.clip(jnp.concatenate([d_low, d_up], axis=-1) / _RADIUS, -1.0, 1.0)
    pt = pt_ref[...]
    onehot = (pt == lax.broadcasted_iota(jnp.int32, (_RB, _NTYPES), 1)
              ).astype(jnp.float32)
    emb = jnp.dot(onehot, emb_ref[...], preferred_element_type=jnp.float32)
    nodes = jnp.concatenate([mr, vel, nd, emb], axis=-1)
    h1 = jnp.maximum(
        jnp.dot(nodes, w1_ref[...], preferred_element_type=jnp.float32)
        + b1_ref[...], 0.0)
    h2 = jnp.dot(h1, w2_ref[...], preferred_element_type=jnp.float32) + b2_ref[...]
    h_ref[...] = _layernorm(h2)


def _upd_body(h_ref, parts_ref, degp_ref, w1h_ref, w1a_ref, b1_ref,
              w2_ref, b2_ref, o_ref):
    p = parts_ref[...]
    dg = degp_ref[...]
    deg = jnp.maximum(dg[0, :, 0:1] + dg[1, :, 0:1], 1.0)
    aggr = (p[0] + p[1]) / deg
    h = h_ref[...]
    u1 = jnp.maximum(
        jnp.dot(h, w1h_ref[...], preferred_element_type=jnp.float32)
        + jnp.dot(aggr, w1a_ref[...], preferred_element_type=jnp.float32)
        + b1_ref[...], 0.0)
    upd = jnp.dot(u1, w2_ref[...], preferred_element_type=jnp.float32) + b2_ref[...]
    o_ref[...] = _layernorm(h + upd)


def _dec_body(h_ref, pos_ref, w1_ref, b1_ref, w2_ref, b2_ref, o_ref):
    h = h_ref[...]
    a1 = jnp.maximum(
        jnp.dot(h, w1_ref[...], preferred_element_type=jnp.float32)
        + b1_ref[...], 0.0)
    acc = jnp.dot(a1, w2_ref[...], preferred_element_type=jnp.float32) + b2_ref[...]
    x = pos_ref[...]
    mr = x[:, 15:18]
    prev = x[:, 12:15]
    o_ref[...] = mr + (mr - prev) + acc


def _full(shape):
    return pl.BlockSpec(shape, lambda i: (0,) * len(shape))


def kernel(position_sequence, type_embedding, W_e1, b_e1, W_e2, b_e2,
           W_p1, b_p1, W_p2, b_p2, W_d1, b_d1, W_d2, b_d2,
           particle_types, edge_index):
    f32 = jnp.float32
    posflat = position_sequence.reshape(_N, 18)
    pt2 = particle_types.reshape(_N, 1)

    pad = _EPAD - _E
    s_p = jnp.concatenate([edge_index[0], jnp.zeros((pad,), jnp.int32)])
    r_p = jnp.concatenate([edge_index[1], jnp.full((pad,), _N, jnp.int32)])
    sgrp = s_p.reshape(_NW, 2, _CHH * _CHUNK)
    rgrp = r_p.reshape(_NW, 2, _CHH * _CHUNK)
    zeros_big = jnp.zeros((_NPAD, _LAT), f32)
    ones_tab = jnp.ones((_N, _LAT), f32)

    h = pl.pallas_call(
        _enc_body,
        grid=(_GRID,),
        in_specs=[
            pl.BlockSpec((_RB, 18), lambda i: (i, 0)),
            pl.BlockSpec((_RB, 1), lambda i: (i, 0)),
            _full((_NTYPES, _EMB)),
            _full((40, _LAT)),
            _full((1, _LAT)),
            _full((_LAT, _LAT)),
            _full((1, _LAT)),
        ],
        out_specs=pl.BlockSpec((_RB, _LAT), lambda i: (i, 0)),
        out_shape=jax.ShapeDtypeStruct((_N, _LAT), f32),
    )(posflat, pt2, type_embedding, W_e1, b_e1.reshape(1, _LAT),
      W_e2, b_e2.reshape(1, _LAT))

    degp = _sc_aggregate(ones_tab, sgrp, rgrp, zeros_big)

    upd_call = pl.pallas_call(
        _upd_body,
        grid=(_GRID,),
        in_specs=[
            pl.BlockSpec((_RB, _LAT), lambda i: (i, 0)),
            pl.BlockSpec((_NC, _RB, _LAT), lambda i: (0, i, 0)),
            pl.BlockSpec((_NC, _RB, _LAT), lambda i: (0, i, 0)),
            _full((_LAT, _LAT)),
            _full((_LAT, _LAT)),
            _full((1, _LAT)),
            _full((_LAT, _LAT)),
            _full((1, _LAT)),
        ],
        out_specs=pl.BlockSpec((_RB, _LAT), lambda i: (i, 0)),
        out_shape=jax.ShapeDtypeStruct((_N, _LAT), f32),
    )
    for st in range(_STEPS):
        parts = _sc_aggregate(h, sgrp, rgrp, zeros_big)
        h = upd_call(h, parts, degp,
                     W_p1[st, :_LAT], W_p1[st, _LAT:],
                     b_p1[st].reshape(1, _LAT),
                     W_p2[st], b_p2[st].reshape(1, _LAT))

    new_position = pl.pallas_call(
        _dec_body,
        grid=(_GRID,),
        in_specs=[
            pl.BlockSpec((_RB, _LAT), lambda i: (i, 0)),
            pl.BlockSpec((_RB, 18), lambda i: (i, 0)),
            _full((_LAT, _LAT)),
            _full((1, _LAT)),
            _full((_LAT, 3)),
            _full((1, 3)),
        ],
        out_specs=pl.BlockSpec((_RB, 3), lambda i: (i, 0)),
        out_shape=jax.ShapeDtypeStruct((_N, 3), f32),
    )(h, posflat, W_d1, b_d1.reshape(1, _LAT), W_d2, b_d2.reshape(1, 3))
    return new_position


# speedup vs baseline: 1.8061x; 1.8061x over previous
import functools

import jax
import jax.numpy as jnp
from jax import lax
from jax.experimental import pallas as pl
from jax.experimental.pallas import tpu as pltpu
from jax.experimental.pallas import tpu_sc as plsc

_N = 10000
_E = 320000
_LAT = 128
_STEPS = 10
_RADIUS = 0.015
_NTYPES = 9
_EMB = 16

_NC = 2
_NS = 16
_NW = _NC * _NS
_CHUNK = 128
_CH = -(-_E // (_NW * _CHUNK))
_EPAD = _NW * _CH * _CHUNK
_NPAD = 10112
_RPT = _NPAD // _NS

_RB = 2000
_GRID = _N // _RB

_sc_mesh = plsc.VectorSubcoreMesh(core_axis_name="c", subcore_axis_name="s",
                                  num_cores=_NC, num_subcores=_NS)


def _make_sc_aggregate(ch):
    @functools.partial(
        pl.kernel,
        out_type=jax.ShapeDtypeStruct((_NC, _NPAD, _LAT), jnp.float32),
        mesh=_sc_mesh,
        scratch_types=[
            pltpu.VMEM((ch, _CHUNK), jnp.int32),
            pltpu.VMEM((ch, _CHUNK), jnp.int32),
            pltpu.VMEM((_CHUNK, _LAT), jnp.float32),
            pltpu.VMEM_SHARED((_NPAD, _LAT), jnp.float32),
            pltpu.SemaphoreType.DMA,
        ],
    )
    def aggregate(h_hbm, sgrp_hbm, rgrp_hbm, zeros_hbm, out_hbm,
                  sidx_v, ridx_v, rows_v, acc_sh, gsem):
        c = lax.axis_index("c")
        s = lax.axis_index("s")
        g = c * _NS + s
        row0 = s * _RPT
        pltpu.sync_copy(zeros_hbm.at[pl.ds(row0, _RPT)],
                        acc_sh.at[pl.ds(row0, _RPT)])
        pltpu.sync_copy(sgrp_hbm.at[g], sidx_v)
        pltpu.sync_copy(rgrp_hbm.at[g], ridx_v)
        plsc.subcore_barrier()

        def body(j, carry):
            pltpu.async_copy(h_hbm.at[sidx_v.at[j]], rows_v, gsem).wait()
            pltpu.sync_copy(rows_v, acc_sh.at[ridx_v.at[j]], add=True)
            return carry

        lax.fori_loop(0, ch, body, 0)
        plsc.subcore_barrier()
        pltpu.sync_copy(acc_sh.at[pl.ds(row0, _RPT)],
                        out_hbm.at[c, pl.ds(row0, _RPT)])

    return aggregate


_sc_aggregate = _make_sc_aggregate(_CH)


@functools.partial(
    pl.kernel,
    out_type=jax.ShapeDtypeStruct((_NC, _NPAD, _LAT), jnp.float32),
    mesh=_sc_mesh,
    scratch_types=[
        pltpu.VMEM((_CH, _CHUNK), jnp.int32),
        pltpu.VMEM((_CHUNK, _LAT), jnp.float32),
        pltpu.VMEM_SHARED((_NPAD, _LAT), jnp.float32),
    ],
)
def _sc_degree(rgrp_hbm, ones_hbm, zeros_hbm, out_hbm,
               ridx_v, ones_v, acc_sh):
    c = lax.axis_index("c")
    s = lax.axis_index("s")
    g = c * _NS + s
    row0 = s * _RPT
    pltpu.sync_copy(zeros_hbm.at[pl.ds(row0, _RPT)],
                    acc_sh.at[pl.ds(row0, _RPT)])
    pltpu.sync_copy(rgrp_hbm.at[g], ridx_v)
    pltpu.sync_copy(ones_hbm, ones_v)
    plsc.subcore_barrier()

    def body(j, carry):
        pltpu.sync_copy(ones_v, acc_sh.at[ridx_v.at[j]], add=True)
        return carry

    lax.fori_loop(0, _CH, body, 0)
    plsc.subcore_barrier()
    pltpu.sync_copy(acc_sh.at[pl.ds(row0, _RPT)],
                    out_hbm.at[c, pl.ds(row0, _RPT)])


def _layernorm(x):
    m = jnp.mean(x, axis=-1, keepdims=True)
    d = x - m
    v = jnp.mean(d * d, axis=-1, keepdims=True)
    return d * lax.rsqrt(v + 1e-6)


def _enc_body(pos_ref, pt_ref, emb_ref, w1_ref, b1_ref, w2_ref, b2_ref, h_ref):
    x = pos_ref[...]
    mr = x[:, 15:18]
    vel = x[:, 3:18] - x[:, 0:15]
    d_low = mr - 0.1
    d_up = 0.9 - mr
    nd = jnp.clip(jnp.concatenate([d_low, d_up], axis=-1) / _RADIUS, -1.0, 1.0)
    pt = pt_ref[...]
    onehot = (pt == lax.broadcasted_iota(jnp.int32, (_RB, _NTYPES), 1)
              ).astype(jnp.float32)
    emb = jnp.dot(onehot, emb_ref[...], preferred_element_type=jnp.float32)
    nodes = jnp.concatenate([mr, vel, nd, emb], axis=-1)
    h1 = jnp.maximum(
        jnp.dot(nodes, w1_ref[...], preferred_element_type=jnp.float32)
        + b1_ref[...], 0.0)
    h2 = jnp.dot(h1, w2_ref[...], preferred_element_type=jnp.float32) + b2_ref[...]
    h_ref[...] = _layernorm(h2)


def _upd_body(h_ref, parts_ref, degp_ref, w1h_ref, w1a_ref, b1_ref,
              w2_ref, b2_ref, o_ref):
    p = parts_ref[...]
    dg = degp_ref[...]
    deg = jnp.maximum(dg[0, :, 0:1] + dg[1, :, 0:1], 1.0)
    aggr = (p[0] + p[1]) / deg
    h = h_ref[...]
    u1 = jnp.maximum(
        jnp.dot(h, w1h_ref[...], preferred_element_type=jnp.float32)
        + jnp.dot(aggr, w1a_ref[...], preferred_element_type=jnp.float32)
        + b1_ref[...], 0.0)
    upd = jnp.dot(u1, w2_ref[...], preferred_element_type=jnp.float32) + b2_ref[...]
    o_ref[...] = _layernorm(h + upd)


def _dec_body(h_ref, pos_ref, w1_ref, b1_ref, w2_ref, b2_ref, o_ref):
    h = h_ref[...]
    a1 = jnp.maximum(
        jnp.dot(h, w1_ref[...], preferred_element_type=jnp.float32)
        + b1_ref[...], 0.0)
    acc = jnp.dot(a1, w2_ref[...], preferred_element_type=jnp.float32) + b2_ref[...]
    x = pos_ref[...]
    mr = x[:, 15:18]
    prev = x[:, 12:15]
    o_ref[...] = mr + (mr - prev) + acc


def _full(shape):
    return pl.BlockSpec(shape, lambda i: (0,) * len(shape))


def kernel(position_sequence, type_embedding, W_e1, b_e1, W_e2, b_e2,
           W_p1, b_p1, W_p2, b_p2, W_d1, b_d1, W_d2, b_d2,
           particle_types, edge_index):
    f32 = jnp.float32
    posflat = position_sequence.reshape(_N, 18)
    pt2 = particle_types.reshape(_N, 1)

    pad = _EPAD - _E
    s_p = jnp.concatenate([edge_index[0], jnp.zeros((pad,), jnp.int32)])
    r_p = jnp.concatenate([edge_index[1], jnp.full((pad,), _N, jnp.int32)])
    sgrp = s_p.reshape(_NW, _CH, _CHUNK)
    rgrp = r_p.reshape(_NW, _CH, _CHUNK)
    zeros_big = jnp.zeros((_NPAD, _LAT), f32)
    ones_rows = jnp.ones((_CHUNK, _LAT), f32)

    h = pl.pallas_call(
        _enc_body,
        grid=(_GRID,),
        in_specs=[
            pl.BlockSpec((_RB, 18), lambda i: (i, 0)),
            pl.BlockSpec((_RB, 1), lambda i: (i, 0)),
            _full((_NTYPES, _EMB)),
            _full((40, _LAT)),
            _full((1, _LAT)),
            _full((_LAT, _LAT)),
            _full((1, _LAT)),
        ],
        out_specs=pl.BlockSpec((_RB, _LAT), lambda i: (i, 0)),
        out_shape=jax.ShapeDtypeStruct((_N, _LAT), f32),
    )(posflat, pt2, type_embedding, W_e1, b_e1.reshape(1, _LAT),
      W_e2, b_e2.reshape(1, _LAT))

    degp = _sc_degree(rgrp, ones_rows, zeros_big)

    upd_call = pl.pallas_call(
        _upd_body,
        grid=(_GRID,),
        in_specs=[
            pl.BlockSpec((_RB, _LAT), lambda i: (i, 0)),
            pl.BlockSpec((_NC, _RB, _LAT), lambda i: (0, i, 0)),
            pl.BlockSpec((_NC, _RB, _LAT), lambda i: (0, i, 0)),
            _full((_LAT, _LAT)),
            _full((_LAT, _LAT)),
            _full((1, _LAT)),
            _full((_LAT, _LAT)),
            _full((1, _LAT)),
        ],
        out_specs=pl.BlockSpec((_RB, _LAT), lambda i: (i, 0)),
        out_shape=jax.ShapeDtypeStruct((_N, _LAT), f32),
    )
    for st in range(_STEPS):
        parts = _sc_aggregate(h, sgrp, rgrp, zeros_big)
        h = upd_call(h, parts, degp,
                     W_p1[st, :_LAT], W_p1[st, _LAT:],
                     b_p1[st].reshape(1, _LAT),
                     W_p2[st], b_p2[st].reshape(1, _LAT))

    new_position = pl.pallas_call(
        _dec_body,
        grid=(_GRID,),
        in_specs=[
            pl.BlockSpec((_RB, _LAT), lambda i: (i, 0)),
            pl.BlockSpec((_RB, 18), lambda i: (i, 0)),
            _full((_LAT, _LAT)),
            _full((1, _LAT)),
            _full((_LAT, 3)),
            _full((1, 3)),
        ],
        out_specs=pl.BlockSpec((_RB, 3), lambda i: (i, 0)),
        out_shape=jax.ShapeDtypeStruct((_N, 3), f32),
    )(h, posflat, W_d1, b_d1.reshape(1, _LAT), W_d2, b_d2.reshape(1, 3))
    return new_position


# speedup vs baseline: 2.0170x; 1.1167x over previous
import functools

import jax
import jax.numpy as jnp
from jax import lax
from jax.experimental import pallas as pl
from jax.experimental.pallas import tpu as pltpu
from jax.experimental.pallas import tpu_sc as plsc

_N = 10000
_E = 320000
_LAT = 128
_STEPS = 10
_RADIUS = 0.015
_NTYPES = 9
_EMB = 16

_NC = 2
_NS = 16
_NW = _NC * _NS
_CHUNK = 64
_CH = -(-_E // (_NW * _CHUNK))
_EPAD = _NW * _CH * _CHUNK
_NPAD = 10112
_RPT = _NPAD // _NS

_RB = 2000
_GRID = _N // _RB

_sc_mesh = plsc.VectorSubcoreMesh(core_axis_name="c", subcore_axis_name="s",
                                  num_cores=_NC, num_subcores=_NS)


def _make_sc_aggregate(ch):
    @functools.partial(
        pl.kernel,
        out_type=jax.ShapeDtypeStruct((_NC, _NPAD, _LAT), jnp.float32),
        mesh=_sc_mesh,
        scratch_types=[
            pltpu.VMEM((ch, _CHUNK), jnp.int32),
            pltpu.VMEM((ch, _CHUNK), jnp.int32),
            pltpu.VMEM((_CHUNK, _LAT), jnp.float32),
            pltpu.VMEM_SHARED((_NPAD, _LAT), jnp.float32),
            pltpu.SemaphoreType.DMA,
        ],
    )
    def aggregate(h_hbm, sgrp_hbm, rgrp_hbm, zeros_hbm, out_hbm,
                  sidx_v, ridx_v, rows_v, acc_sh, gsem):
        c = lax.axis_index("c")
        s = lax.axis_index("s")
        g = c * _NS + s
        row0 = s * _RPT
        pltpu.sync_copy(zeros_hbm.at[pl.ds(row0, _RPT)],
                        acc_sh.at[pl.ds(row0, _RPT)])
        pltpu.sync_copy(sgrp_hbm.at[g], sidx_v)
        pltpu.sync_copy(rgrp_hbm.at[g], ridx_v)
        plsc.subcore_barrier()

        def body(j, carry):
            pltpu.async_copy(h_hbm.at[sidx_v.at[j]], rows_v, gsem).wait()
            pltpu.sync_copy(rows_v, acc_sh.at[ridx_v.at[j]], add=True)
            return carry

        lax.fori_loop(0, ch, body, 0)
        plsc.subcore_barrier()
        pltpu.sync_copy(acc_sh.at[pl.ds(row0, _RPT)],
                        out_hbm.at[c, pl.ds(row0, _RPT)])

    return aggregate


_sc_aggregate = _make_sc_aggregate(_CH)


@functools.partial(
    pl.kernel,
    out_type=jax.ShapeDtypeStruct((_NC, _NPAD, _LAT), jnp.float32),
    mesh=_sc_mesh,
    scratch_types=[
        pltpu.VMEM((_CH, _CHUNK), jnp.int32),
        pltpu.VMEM((_CHUNK, _LAT), jnp.float32),
        pltpu.VMEM_SHARED((_NPAD, _LAT), jnp.float32),
    ],
)
def _sc_degree(rgrp_hbm, ones_hbm, zeros_hbm, out_hbm,
               ridx_v, ones_v, acc_sh):
    c = lax.axis_index("c")
    s = lax.axis_index("s")
    g = c * _NS + s
    row0 = s * _RPT
    pltpu.sync_copy(zeros_hbm.at[pl.ds(row0, _RPT)],
                    acc_sh.at[pl.ds(row0, _RPT)])
    pltpu.sync_copy(rgrp_hbm.at[g], ridx_v)
    pltpu.sync_copy(ones_hbm, ones_v)
    plsc.subcore_barrier()

    def body(j, carry):
        pltpu.sync_copy(ones_v, acc_sh.at[ridx_v.at[j]], add=True)
        return carry

    lax.fori_loop(0, _CH, body, 0)
    plsc.subcore_barrier()
    pltpu.sync_copy(acc_sh.at[pl.ds(row0, _RPT)],
                    out_hbm.at[c, pl.ds(row0, _RPT)])


def _layernorm(x):
    m = jnp.mean(x, axis=-1, keepdims=True)
    d = x - m
    v = jnp.mean(d * d, axis=-1, keepdims=True)
    return d * lax.rsqrt(v + 1e-6)


def _enc_body(pos_ref, pt_ref, emb_ref, w1_ref, b1_ref, w2_ref, b2_ref, h_ref):
    x = pos_ref[...]
    mr = x[:, 15:18]
    vel = x[:, 3:18] - x[:, 0:15]
    d_low = mr - 0.1
    d_up = 0.9 - mr
    nd = jnp.clip(jnp.concatenate([d_low, d_up], axis=-1) / _RADIUS, -1.0, 1.0)
    pt = pt_ref[...]
    onehot = (pt == lax.broadcasted_iota(jnp.int32, (_RB, _NTYPES), 1)
              ).astype(jnp.float32)
    emb = jnp.dot(onehot, emb_ref[...], preferred_element_type=jnp.float32)
    nodes = jnp.concatenate([mr, vel, nd, emb], axis=-1)
    h1 = jnp.maximum(
        jnp.dot(nodes, w1_ref[...], preferred_element_type=jnp.float32)
        + b1_ref[...], 0.0)
    h2 = jnp.dot(h1, w2_ref[...], preferred_element_type=jnp.float32) + b2_ref[...]
    h_ref[...] = _layernorm(h2)


def _upd_body(h_ref, parts_ref, degp_ref, w1h_ref, w1a_ref, b1_ref,
              w2_ref, b2_ref, o_ref):
    p = parts_ref[...]
    dg = degp_ref[...]
    deg = jnp.maximum(dg[0, :, 0:1] + dg[1, :, 0:1], 1.0)
    aggr = (p[0] + p[1]) / deg
    h = h_ref[...]
    u1 = jnp.maximum(
        jnp.dot(h, w1h_ref[...], preferred_element_type=jnp.float32)
        + jnp.dot(aggr, w1a_ref[...], preferred_element_type=jnp.float32)
        + b1_ref[...], 0.0)
    upd = jnp.dot(u1, w2_ref[...], preferred_element_type=jnp.float32) + b2_ref[...]
    o_ref[...] = _layernorm(h + upd)


def _dec_body(h_ref, pos_ref, w1_ref, b1_ref, w2_ref, b2_ref, o_ref):
    h = h_ref[...]
    a1 = jnp.maximum(
        jnp.dot(h, w1_ref[...], preferred_element_type=jnp.float32)
        + b1_ref[...], 0.0)
    acc = jnp.dot(a1, w2_ref[...], preferred_element_type=jnp.float32) + b2_ref[...]
    x = pos_ref[...]
    mr = x[:, 15:18]
    prev = x[:, 12:15]
    o_ref[...] = mr + (mr - prev) + acc


def _full(shape):
    return pl.BlockSpec(shape, lambda i: (0,) * len(shape))


def kernel(position_sequence, type_embedding, W_e1, b_e1, W_e2, b_e2,
           W_p1, b_p1, W_p2, b_p2, W_d1, b_d1, W_d2, b_d2,
           particle_types, edge_index):
    f32 = jnp.float32
    posflat = position_sequence.reshape(_N, 18)
    pt2 = particle_types.reshape(_N, 1)

    pad = _EPAD - _E
    s_p = jnp.concatenate([edge_index[0], jnp.zeros((pad,), jnp.int32)])
    r_p = jnp.concatenate([edge_index[1], jnp.full((pad,), _N, jnp.int32)])
    sgrp = s_p.reshape(_NW, _CH, _CHUNK)
    rgrp = r_p.reshape(_NW, _CH, _CHUNK)
    zeros_big = jnp.zeros((_NPAD, _LAT), f32)
    ones_rows = jnp.ones((_CHUNK, _LAT), f32)

    h = pl.pallas_call(
        _enc_body,
        grid=(_GRID,),
        in_specs=[
            pl.BlockSpec((_RB, 18), lambda i: (i, 0)),
            pl.BlockSpec((_RB, 1), lambda i: (i, 0)),
            _full((_NTYPES, _EMB)),
            _full((40, _LAT)),
            _full((1, _LAT)),
            _full((_LAT, _LAT)),
            _full((1, _LAT)),
        ],
        out_specs=pl.BlockSpec((_RB, _LAT), lambda i: (i, 0)),
        out_shape=jax.ShapeDtypeStruct((_N, _LAT), f32),
    )(posflat, pt2, type_embedding, W_e1, b_e1.reshape(1, _LAT),
      W_e2, b_e2.reshape(1, _LAT))

    degp = _sc_degree(rgrp, ones_rows, zeros_big)

    upd_call = pl.pallas_call(
        _upd_body,
        grid=(_GRID,),
        in_specs=[
            pl.BlockSpec((_RB, _LAT), lambda i: (i, 0)),
            pl.BlockSpec((_NC, _RB, _LAT), lambda i: (0, i, 0)),
            pl.BlockSpec((_NC, _RB, _LAT), lambda i: (0, i, 0)),
            _full((_LAT, _LAT)),
            _full((_LAT, _LAT)),
            _full((1, _LAT)),
            _full((_LAT, _LAT)),
            _full((1, _LAT)),
        ],
        out_specs=pl.BlockSpec((_RB, _LAT), lambda i: (i, 0)),
        out_shape=jax.ShapeDtypeStruct((_N, _LAT), f32),
    )
    for st in range(_STEPS):
        parts = _sc_aggregate(h, sgrp, rgrp, zeros_big)
        h = upd_call(h, parts, degp,
                     W_p1[st, :_LAT], W_p1[st, _LAT:],
                     b_p1[st].reshape(1, _LAT),
                     W_p2[st], b_p2[st].reshape(1, _LAT))

    new_position = pl.pallas_call(
        _dec_body,
        grid=(_GRID,),
        in_specs=[
            pl.BlockSpec((_RB, _LAT), lambda i: (i, 0)),
            pl.BlockSpec((_RB, 18), lambda i: (i, 0)),
            _full((_LAT, _LAT)),
            _full((1, _LAT)),
            _full((_LAT, 3)),
            _full((1, 3)),
        ],
        out_specs=pl.BlockSpec((_RB, 3), lambda i: (i, 0)),
        out_shape=jax.ShapeDtypeStruct((_N, 3), f32),
    )(h, posflat, W_d1, b_d1.reshape(1, _LAT), W_d2, b_d2.reshape(1, 3))
    return new_position
